# Initial kernel scaffold; baseline (speedup 1.0000x reference)
import jax
import jax.numpy as jnp
from jax.experimental import pallas as pl


def kernel(inputs_embeds, position_embeddings, special_embeddings, ln_gamma, ln_beta):
    raise NotImplementedError("write your pallas kernel here")


# speedup vs baseline: 1.8890x; 1.8890x over previous
import functools

import jax
import jax.numpy as jnp
from jax.experimental import pallas as pl
from jax.experimental.pallas import tpu as pltpu

_NUM_SPECIAL = 2
_EPS = 1e-12
_BLK = 256


def _body(in_ref, pos_ref, special_ref, gamma_ref, beta_ref, out_ref, carry_ref):
    i = pl.program_id(0)
    b = pl.program_id(1)
    cur = in_ref[0]

    prev2 = jnp.where(i == 0, special_ref[...], carry_ref[b])
    x = jnp.concatenate([prev2, cur[: _BLK - _NUM_SPECIAL]], axis=0)
    x = x + pos_ref[...]

    mean = jnp.mean(x, axis=-1, keepdims=True)
    centered = x - mean
    var = jnp.mean(centered * centered, axis=-1, keepdims=True)
    normed = centered * jax.lax.rsqrt(var + _EPS)
    out_ref[0] = normed * gamma_ref[...] + beta_ref[...]

    carry_ref[b] = cur[_BLK - _NUM_SPECIAL :]


def kernel(inputs_embeds, position_embeddings, special_embeddings, ln_gamma, ln_beta):
    bs, seq_in, hidden = inputs_embeds.shape
    seq_out = seq_in + _NUM_SPECIAL
    n_in_blocks = seq_in // _BLK
    n_blocks = pl.cdiv(seq_out, _BLK)

    gamma2d = ln_gamma.reshape(1, hidden)
    beta2d = ln_beta.reshape(1, hidden)

    grid = (n_blocks, bs)

    return pl.pallas_call(
        _body,
        grid=grid,
        in_specs=[
            pl.BlockSpec(
                (1, _BLK, hidden),
                lambda i, b: (b, jnp.minimum(i, n_in_blocks - 1), 0),
            ),
            pl.BlockSpec((_BLK, hidden), lambda i, b: (i, 0)),
            pl.BlockSpec((_NUM_SPECIAL, hidden), lambda i, b: (0, 0)),
            pl.BlockSpec((1, hidden), lambda i, b: (0, 0)),
            pl.BlockSpec((1, hidden), lambda i, b: (0, 0)),
        ],
        out_specs=pl.BlockSpec((1, _BLK, hidden), lambda i, b: (b, i, 0)),
        out_shape=jax.ShapeDtypeStruct((bs, seq_out, hidden), jnp.float32),
        scratch_shapes=[pltpu.VMEM((bs, _NUM_SPECIAL, hidden), jnp.float32)],
        compiler_params=pltpu.CompilerParams(
            dimension_semantics=("arbitrary", "arbitrary"),
        ),
    )(inputs_embeds, position_embeddings, special_embeddings, gamma2d, beta2d)
